# baseline (speedup 1.0000x reference)
import jax
import jax.numpy as jnp
from jax import lax
from jax.experimental import pallas as pl
from jax.experimental.pallas import tpu as pltpu
from jax.experimental.pallas import tpu_sc as plsc

_N = 10000
_E = 160000
_D = 128
_R = 5
_NB = 8
_SUB = _D // _NB

_NC = 2
_NS = 16
_NW = _NC * _NS

_CHUNK = 128
_EPW = 5120
_NCHUNK = _EPW // _CHUNK
_E_PAD = _NW * _EPW
_NCH_TOT = _E_PAD // _CHUNK
_CA = 52
_CB = (_NCH_TOT - 16 * _CA) // 16
_N_PAD = 10112
_RPT = _N_PAD // _NS


def _agg_body(table, gidx, zb128, agg_out, agg_sh, idx_v, rows_v, zb_v, sem):
  c = lax.axis_index("c")
  s = lax.axis_index("s")

  start = jnp.where(c == 0, s * _CA, 16 * _CA + s * _CB)
  nch = jnp.where(c == 0, _CA, _CB)
  pltpu.sync_copy(gidx.at[pl.ds(start, _CA)], idx_v)
  pltpu.sync_copy(zb128, zb_v)

  base = s * _RPT

  def _zero(t, carry):
    pltpu.sync_copy(zb_v, agg_sh.at[pl.ds(base + t * 8, 8)])
    return carry

  lax.fori_loop(0, _RPT // 8, _zero, 0)
  plsc.subcore_barrier()

  def _edge(j, carry):
    pltpu.async_copy(table.at[idx_v.at[j, 0]], rows_v, sem).wait()
    pltpu.sync_copy(rows_v, agg_sh.at[idx_v.at[j, 1]], add=True)
    return carry

  lax.fori_loop(0, nch, _edge, 0)
  plsc.subcore_barrier()

  pltpu.sync_copy(agg_sh.at[pl.ds(base, _RPT)],
                  agg_out.at[pl.ds(c * _N_PAD + base, _RPT)])


def _deg_body(dsti, zb128, ones128, deg_out, deg_sh, dst_v, ones_v, zb_v):
  c = lax.axis_index("c")
  s = lax.axis_index("s")
  wid = c * _NS + s
  pltpu.sync_copy(dsti.at[wid], dst_v)
  pltpu.sync_copy(zb128, zb_v)
  pltpu.sync_copy(ones128, ones_v)
  base = s * _RPT

  def _zero(t, carry):
    pltpu.sync_copy(zb_v, deg_sh.at[pl.ds(base + t * 8, 8)])
    return carry

  lax.fori_loop(0, _RPT // 8, _zero, 0)
  plsc.subcore_barrier()

  def _edge(j, carry):
    pltpu.sync_copy(ones_v, deg_sh.at[dst_v.at[j]], add=True)
    return carry

  lax.fori_loop(0, _NCHUNK, _edge, 0)
  plsc.subcore_barrier()
  pltpu.sync_copy(deg_sh.at[pl.ds(base, _RPT)],
                  deg_out.at[pl.ds(c * _N_PAD + base, _RPT)])


def _sc_mesh():
  return plsc.VectorSubcoreMesh(core_axis_name="c", subcore_axis_name="s",
                                num_cores=_NC, num_subcores=_NS)


_sc_cache = {}


def _get_agg():
  if "agg" not in _sc_cache:
    _sc_cache["agg"] = pl.kernel(
        _agg_body,
        out_type=[jax.ShapeDtypeStruct((_NC * _N_PAD, _D), jnp.float32)],
        mesh=_sc_mesh(),
        scratch_types=[
            pltpu.VMEM_SHARED((_N_PAD, _D), jnp.float32),
            pltpu.VMEM((_CA, 2, _CHUNK), jnp.int32),
            pltpu.VMEM((_CHUNK, _D), jnp.float32),
            pltpu.VMEM((8, _D), jnp.float32),
            pltpu.SemaphoreType.DMA,
        ],
    )
  return _sc_cache["agg"]


def _get_deg():
  if "deg" not in _sc_cache:
    _sc_cache["deg"] = pl.kernel(
        _deg_body,
        out_type=[jax.ShapeDtypeStruct((_NC * _N_PAD, _D), jnp.float32)],
        mesh=_sc_mesh(),
        scratch_types=[
            pltpu.VMEM_SHARED((_N_PAD, _D), jnp.float32),
            pltpu.VMEM((_NCHUNK, _CHUNK), jnp.int32),
            pltpu.VMEM((_CHUNK, _D), jnp.float32),
            pltpu.VMEM((8, _D), jnp.float32),
        ],
    )
  return _sc_cache["deg"]


_BN = 1264


def _mm_body(x_ref, w_ref, o_ref):
  o_ref[...] = jnp.dot(x_ref[...], w_ref[...],
                       preferred_element_type=jnp.float32)


def _tc_table(h_pad, wcat):
  return pl.pallas_call(
      _mm_body,
      grid=(_N_PAD // _BN,),
      in_specs=[
          pl.BlockSpec((_BN, _D), lambda i: (i, 0)),
          pl.BlockSpec((_D, _R * _D), lambda i: (0, 0)),
      ],
      out_specs=pl.BlockSpec((_BN, _R * _D), lambda i: (i, 0)),
      out_shape=jax.ShapeDtypeStruct((_N_PAD, _R * _D), jnp.float32),
  )(h_pad, wcat)


def _layer_body(h_ref, a0_ref, a1_ref, d0_ref, d1_ref, lw_ref, b_ref,
                wc_ref, h1_ref, ht_ref):
  agg = a0_ref[...] + a1_ref[...]
  deg = d0_ref[:, 0:1] + d1_ref[:, 0:1]
  norm = jnp.where(deg > 0.0, 1.0 / jnp.maximum(deg, 1.0), 0.0)
  z = agg * norm + jnp.dot(h_ref[...], lw_ref[...],
                           preferred_element_type=jnp.float32) + b_ref[...]
  h1 = jnp.maximum(z, 0.0)
  h1_ref[...] = h1
  ht_ref[...] = jnp.dot(h1, wc_ref[...], preferred_element_type=jnp.float32)


def _tc_layer_mid(h_pad, a0, a1, d0, d1, loop_w, b, wcat_next):
  return pl.pallas_call(
      _layer_body,
      grid=(_N_PAD // _BN,),
      in_specs=[
          pl.BlockSpec((_BN, _D), lambda i: (i, 0)),
          pl.BlockSpec((_BN, _D), lambda i: (i, 0)),
          pl.BlockSpec((_BN, _D), lambda i: (i, 0)),
          pl.BlockSpec((_BN, _D), lambda i: (i, 0)),
          pl.BlockSpec((_BN, _D), lambda i: (i, 0)),
          pl.BlockSpec((_D, _D), lambda i: (0, 0)),
          pl.BlockSpec((1, _D), lambda i: (0, 0)),
          pl.BlockSpec((_D, _R * _D), lambda i: (0, 0)),
      ],
      out_specs=[
          pl.BlockSpec((_BN, _D), lambda i: (i, 0)),
          pl.BlockSpec((_BN, _R * _D), lambda i: (i, 0)),
      ],
      out_shape=[
          jax.ShapeDtypeStruct((_N_PAD, _D), jnp.float32),
          jax.ShapeDtypeStruct((_N_PAD, _R * _D), jnp.float32),
      ],
  )(h_pad, a0, a1, d0, d1, loop_w, b, wcat_next)


def _final_body(h_ref, a0_ref, a1_ref, d0_ref, d1_ref, lw_ref, b_ref, o_ref):
  agg = a0_ref[...] + a1_ref[...]
  deg = d0_ref[:, 0:1] + d1_ref[:, 0:1]
  norm = jnp.where(deg > 0.0, 1.0 / jnp.maximum(deg, 1.0), 0.0)
  o_ref[...] = agg * norm + jnp.dot(h_ref[...], lw_ref[...],
                                    preferred_element_type=jnp.float32) + b_ref[...]


def _tc_layer_final(h_pad, a0, a1, d0, d1, loop_w, b):
  return pl.pallas_call(
      _final_body,
      grid=(_N_PAD // _BN,),
      in_specs=[
          pl.BlockSpec((_BN, _D), lambda i: (i, 0)),
          pl.BlockSpec((_BN, _D), lambda i: (i, 0)),
          pl.BlockSpec((_BN, _D), lambda i: (i, 0)),
          pl.BlockSpec((_BN, _D), lambda i: (i, 0)),
          pl.BlockSpec((_BN, _D), lambda i: (i, 0)),
          pl.BlockSpec((_D, _D), lambda i: (0, 0)),
          pl.BlockSpec((1, _D), lambda i: (0, 0)),
      ],
      out_specs=pl.BlockSpec((_BN, _D), lambda i: (i, 0)),
      out_shape=jax.ShapeDtypeStruct((_N_PAD, _D), jnp.float32),
  )(h_pad, a0, a1, d0, d1, loop_w, b)


def _blockdiag_cat(W):
  Wd = jnp.zeros((_R, _D, _D), W.dtype)
  for b in range(_NB):
    Wd = Wd.at[:, b * _SUB:(b + 1) * _SUB, b * _SUB:(b + 1) * _SUB].set(W[:, b])
  return Wd.transpose(1, 0, 2).reshape(_D, _R * _D)


@jax.jit
def kernel(h, edge_index, e_feat, W0, loop_w0, b0, W1, loop_w1, b1):
  src = edge_index[0].astype(jnp.int32)
  dst = edge_index[1].astype(jnp.int32)
  ef = e_feat.astype(jnp.int32)

  pad = _E_PAD - _E
  gidx = jnp.pad(src * _R + ef, (0, pad)).reshape(_NCH_TOT, 1, _CHUNK)
  dstp = jnp.pad(dst, (0, pad), constant_values=_N)
  dsti = dstp.reshape(_NCH_TOT, 1, _CHUNK)
  gd = jnp.concatenate([gidx, dsti], axis=1)
  gd = jnp.pad(gd, ((0, _CA - _CB), (0, 0), (0, 0)))
  dsti3 = dstp.reshape(_NW, _NCHUNK, _CHUNK)

  zb128 = jnp.zeros((8, _D), jnp.float32)
  ones128 = jnp.ones((_CHUNK, _D), jnp.float32)

  h_pad = jnp.pad(h, ((0, _N_PAD - _N), (0, 0)))
  wcat0 = _blockdiag_cat(W0)
  wcat1 = _blockdiag_cat(W1)

  (degp,) = _get_deg()(dsti3, zb128, ones128)
  d0 = degp[:_N_PAD]
  d1 = degp[_N_PAD:]

  ht0 = _tc_table(h_pad, wcat0).reshape(_N_PAD * _R, _D)
  (aggp0,) = _get_agg()(ht0, gd, zb128)
  h1_pad, ht1 = _tc_layer_mid(h_pad, aggp0[:_N_PAD], aggp0[_N_PAD:], d0, d1,
                              loop_w0, b0.reshape(1, _D), wcat1)

  (aggp1,) = _get_agg()(ht1.reshape(_N_PAD * _R, _D), gd, zb128)
  out = _tc_layer_final(h1_pad, aggp1[:_N_PAD], aggp1[_N_PAD:], d0, d1,
                        loop_w1, b1.reshape(1, _D))
  return out[:_N]


# speedup vs baseline: 1.1936x; 1.1936x over previous
import jax
import jax.numpy as jnp
from jax import lax
from jax.experimental import pallas as pl
from jax.experimental.pallas import tpu as pltpu
from jax.experimental.pallas import tpu_sc as plsc

_N = 10000
_E = 160000
_D = 128
_R = 5
_NB = 8
_SUB = _D // _NB

_NC = 2
_NS = 16
_NW = _NC * _NS

_CHUNK = 128
_EPW = 5120
_NCHUNK = _EPW // _CHUNK
_E_PAD = _NW * _EPW
_GC = 64
_NSUB = 4
_NCH_TOT = _E_PAD // _GC
_CA = 120
_CB = (_NCH_TOT - 16 * _CA) // 16
_N_PAD = 10112
_RPT = _N_PAD // _NS


def _agg_body(table, gidx, zb128, agg_out, agg_sh, idx_v, rows_v, zb_v, *sems):
  c = lax.axis_index("c")
  s = lax.axis_index("s")

  start = jnp.where(c == 0, s * _CA, 16 * _CA + s * _CB)
  nch = jnp.where(c == 0, _CA, _CB)
  pltpu.sync_copy(gidx.at[pl.ds(start, _CA)], idx_v)
  pltpu.sync_copy(zb128, zb_v)

  base = s * _RPT

  def _zero(t, carry):
    pltpu.sync_copy(zb_v, agg_sh.at[pl.ds(base + t * 8, 8)])
    return carry

  lax.fori_loop(0, _RPT // 8, _zero, 0)
  plsc.subcore_barrier()

  sub = _GC // _NSUB

  def fire(j, b):
    for k in range(_NSUB):
      pltpu.async_copy(table.at[idx_v.at[j, 0, pl.ds(k * sub, sub)]],
                       rows_v.at[b, pl.ds(k * sub, sub)],
                       sems[b * _NSUB + k])

  def drain_scatter(j, b):
    for k in range(_NSUB):
      pltpu.make_async_copy(table.at[idx_v.at[j, 0, pl.ds(k * sub, sub)]],
                            rows_v.at[b, pl.ds(k * sub, sub)],
                            sems[b * _NSUB + k]).wait()
    pltpu.sync_copy(rows_v.at[b], agg_sh.at[idx_v.at[j, 1]], add=True)

  fire(0, 0)

  def _pair(p, carry):
    fire(2 * p + 1, 1)
    drain_scatter(2 * p, 0)

    @pl.when(2 * p + 2 < nch)
    def _():
      fire(2 * p + 2, 0)

    drain_scatter(2 * p + 1, 1)
    return carry

  lax.fori_loop(0, nch // 2, _pair, 0)
  plsc.subcore_barrier()

  pltpu.sync_copy(agg_sh.at[pl.ds(base, _RPT)],
                  agg_out.at[pl.ds(c * _N_PAD + base, _RPT)])


def _deg_body(dsti, zb128, ones128, deg_out, deg_sh, dst_v, ones_v, zb_v):
  c = lax.axis_index("c")
  s = lax.axis_index("s")
  wid = c * _NS + s
  pltpu.sync_copy(dsti.at[wid], dst_v)
  pltpu.sync_copy(zb128, zb_v)
  pltpu.sync_copy(ones128, ones_v)
  base = s * _RPT

  def _zero(t, carry):
    pltpu.sync_copy(zb_v, deg_sh.at[pl.ds(base + t * 8, 8)])
    return carry

  lax.fori_loop(0, _RPT // 8, _zero, 0)
  plsc.subcore_barrier()

  def _edge(j, carry):
    pltpu.sync_copy(ones_v, deg_sh.at[dst_v.at[j]], add=True)
    return carry

  lax.fori_loop(0, _NCHUNK, _edge, 0)
  plsc.subcore_barrier()
  pltpu.sync_copy(deg_sh.at[pl.ds(base, _RPT)],
                  deg_out.at[pl.ds(c * _N_PAD + base, _RPT)])


def _sc_mesh():
  return plsc.VectorSubcoreMesh(core_axis_name="c", subcore_axis_name="s",
                                num_cores=_NC, num_subcores=_NS)


_sc_cache = {}


def _get_agg():
  if "agg" not in _sc_cache:
    _sc_cache["agg"] = pl.kernel(
        _agg_body,
        out_type=[jax.ShapeDtypeStruct((_NC * _N_PAD, _D), jnp.float32)],
        mesh=_sc_mesh(),
        scratch_types=[
            pltpu.VMEM_SHARED((_N_PAD, _D), jnp.float32),
            pltpu.VMEM((_CA, 2, _GC), jnp.int32),
            pltpu.VMEM((2, _GC, _D), jnp.float32),
            pltpu.VMEM((8, _D), jnp.float32),
        ] + [pltpu.SemaphoreType.DMA] * (2 * _NSUB),
    )
  return _sc_cache["agg"]


def _get_deg():
  if "deg" not in _sc_cache:
    _sc_cache["deg"] = pl.kernel(
        _deg_body,
        out_type=[jax.ShapeDtypeStruct((_NC * _N_PAD, _D), jnp.float32)],
        mesh=_sc_mesh(),
        scratch_types=[
            pltpu.VMEM_SHARED((_N_PAD, _D), jnp.float32),
            pltpu.VMEM((_NCHUNK, _CHUNK), jnp.int32),
            pltpu.VMEM((_CHUNK, _D), jnp.float32),
            pltpu.VMEM((8, _D), jnp.float32),
        ],
    )
  return _sc_cache["deg"]


_BN = 1264


def _mm_body(x_ref, w_ref, o_ref):
  o_ref[...] = jnp.dot(x_ref[...], w_ref[...],
                       preferred_element_type=jnp.float32)


def _tc_table(h_pad, wcat):
  return pl.pallas_call(
      _mm_body,
      grid=(_N_PAD // _BN,),
      in_specs=[
          pl.BlockSpec((_BN, _D), lambda i: (i, 0)),
          pl.BlockSpec((_D, _R * _D), lambda i: (0, 0)),
      ],
      out_specs=pl.BlockSpec((_BN, _R * _D), lambda i: (i, 0)),
      out_shape=jax.ShapeDtypeStruct((_N_PAD, _R * _D), jnp.float32),
  )(h_pad, wcat)


def _layer_body(h_ref, a0_ref, a1_ref, d0_ref, d1_ref, lw_ref, b_ref,
                wc_ref, h1_ref, ht_ref):
  agg = a0_ref[...] + a1_ref[...]
  deg = d0_ref[:, 0:1] + d1_ref[:, 0:1]
  norm = jnp.where(deg > 0.0, 1.0 / jnp.maximum(deg, 1.0), 0.0)
  z = agg * norm + jnp.dot(h_ref[...], lw_ref[...],
                           preferred_element_type=jnp.float32) + b_ref[...]
  h1 = jnp.maximum(z, 0.0)
  h1_ref[...] = h1
  ht_ref[...] = jnp.dot(h1, wc_ref[...], preferred_element_type=jnp.float32)


def _tc_layer_mid(h_pad, a0, a1, d0, d1, loop_w, b, wcat_next):
  return pl.pallas_call(
      _layer_body,
      grid=(_N_PAD // _BN,),
      in_specs=[
          pl.BlockSpec((_BN, _D), lambda i: (i, 0)),
          pl.BlockSpec((_BN, _D), lambda i: (i, 0)),
          pl.BlockSpec((_BN, _D), lambda i: (i, 0)),
          pl.BlockSpec((_BN, _D), lambda i: (i, 0)),
          pl.BlockSpec((_BN, _D), lambda i: (i, 0)),
          pl.BlockSpec((_D, _D), lambda i: (0, 0)),
          pl.BlockSpec((1, _D), lambda i: (0, 0)),
          pl.BlockSpec((_D, _R * _D), lambda i: (0, 0)),
      ],
      out_specs=[
          pl.BlockSpec((_BN, _D), lambda i: (i, 0)),
          pl.BlockSpec((_BN, _R * _D), lambda i: (i, 0)),
      ],
      out_shape=[
          jax.ShapeDtypeStruct((_N_PAD, _D), jnp.float32),
          jax.ShapeDtypeStruct((_N_PAD, _R * _D), jnp.float32),
      ],
  )(h_pad, a0, a1, d0, d1, loop_w, b, wcat_next)


def _final_body(h_ref, a0_ref, a1_ref, d0_ref, d1_ref, lw_ref, b_ref, o_ref):
  agg = a0_ref[...] + a1_ref[...]
  deg = d0_ref[:, 0:1] + d1_ref[:, 0:1]
  norm = jnp.where(deg > 0.0, 1.0 / jnp.maximum(deg, 1.0), 0.0)
  o_ref[...] = agg * norm + jnp.dot(h_ref[...], lw_ref[...],
                                    preferred_element_type=jnp.float32) + b_ref[...]


def _tc_layer_final(h_pad, a0, a1, d0, d1, loop_w, b):
  return pl.pallas_call(
      _final_body,
      grid=(_N_PAD // _BN,),
      in_specs=[
          pl.BlockSpec((_BN, _D), lambda i: (i, 0)),
          pl.BlockSpec((_BN, _D), lambda i: (i, 0)),
          pl.BlockSpec((_BN, _D), lambda i: (i, 0)),
          pl.BlockSpec((_BN, _D), lambda i: (i, 0)),
          pl.BlockSpec((_BN, _D), lambda i: (i, 0)),
          pl.BlockSpec((_D, _D), lambda i: (0, 0)),
          pl.BlockSpec((1, _D), lambda i: (0, 0)),
      ],
      out_specs=pl.BlockSpec((_BN, _D), lambda i: (i, 0)),
      out_shape=jax.ShapeDtypeStruct((_N_PAD, _D), jnp.float32),
  )(h_pad, a0, a1, d0, d1, loop_w, b)


def _blockdiag_cat(W):
  Wd = jnp.zeros((_R, _D, _D), W.dtype)
  for b in range(_NB):
    Wd = Wd.at[:, b * _SUB:(b + 1) * _SUB, b * _SUB:(b + 1) * _SUB].set(W[:, b])
  return Wd.transpose(1, 0, 2).reshape(_D, _R * _D)


@jax.jit
def kernel(h, edge_index, e_feat, W0, loop_w0, b0, W1, loop_w1, b1):
  src = edge_index[0].astype(jnp.int32)
  dst = edge_index[1].astype(jnp.int32)
  ef = e_feat.astype(jnp.int32)

  pad = _E_PAD - _E
  gidx = jnp.pad(src * _R + ef, (0, pad)).reshape(_NCH_TOT, 1, _GC)
  dstp = jnp.pad(dst, (0, pad), constant_values=_N)
  dsti = dstp.reshape(_NCH_TOT, 1, _GC)
  gd = jnp.concatenate([gidx, dsti], axis=1)
  gd = jnp.pad(gd, ((0, _CA - _CB), (0, 0), (0, 0)))
  dsti3 = dstp.reshape(_NW, _NCHUNK, _CHUNK)

  zb128 = jnp.zeros((8, _D), jnp.float32)
  ones128 = jnp.ones((_CHUNK, _D), jnp.float32)

  h_pad = jnp.pad(h, ((0, _N_PAD - _N), (0, 0)))
  wcat0 = _blockdiag_cat(W0)
  wcat1 = _blockdiag_cat(W1)

  (degp,) = _get_deg()(dsti3, zb128, ones128)
  d0 = degp[:_N_PAD]
  d1 = degp[_N_PAD:]

  ht0 = _tc_table(h_pad, wcat0).reshape(_N_PAD * _R, _D)
  (aggp0,) = _get_agg()(ht0, gd, zb128)
  h1_pad, ht1 = _tc_layer_mid(h_pad, aggp0[:_N_PAD], aggp0[_N_PAD:], d0, d1,
                              loop_w0, b0.reshape(1, _D), wcat1)

  (aggp1,) = _get_agg()(ht1.reshape(_N_PAD * _R, _D), gd, zb128)
  out = _tc_layer_final(h1_pad, aggp1[:_N_PAD], aggp1[_N_PAD:], d0, d1,
                        loop_w1, b1.reshape(1, _D))
  return out[:_N]


# speedup vs baseline: 1.1956x; 1.0017x over previous
import jax
import jax.numpy as jnp
from jax import lax
from jax.experimental import pallas as pl
from jax.experimental.pallas import tpu as pltpu
from jax.experimental.pallas import tpu_sc as plsc

_N = 10000
_E = 160000
_D = 128
_R = 5
_NB = 8
_SUB = _D // _NB

_NC = 2
_NS = 16
_NW = _NC * _NS

_CHUNK = 128
_EPW = 5120
_NCHUNK = _EPW // _CHUNK
_E_PAD = _NW * _EPW
_GC = 64
_NSUB = 4
_NCH_TOT = _E_PAD // _GC
_CA = 128
_CB = (_NCH_TOT - 16 * _CA) // 16
_N_PAD = 10112
_RPT = _N_PAD // _NS


def _agg_body(table, gidx, zb128, agg_out, agg_sh, idx_v, rows_v, zb_v, *sems):
  c = lax.axis_index("c")
  s = lax.axis_index("s")

  start = jnp.where(c == 0, s * _CA, 16 * _CA + s * _CB)
  nch = jnp.where(c == 0, _CA, _CB)
  pltpu.sync_copy(gidx.at[pl.ds(start, _CA)], idx_v)
  pltpu.sync_copy(zb128, zb_v)

  base = s * _RPT

  def _zero(t, carry):
    pltpu.sync_copy(zb_v, agg_sh.at[pl.ds(base + t * 8, 8)])
    return carry

  lax.fori_loop(0, _RPT // 8, _zero, 0)
  plsc.subcore_barrier()

  sub = _GC // _NSUB

  def fire(j, b):
    for k in range(_NSUB):
      pltpu.async_copy(table.at[idx_v.at[j, 0, pl.ds(k * sub, sub)]],
                       rows_v.at[b, pl.ds(k * sub, sub)],
                       sems[b * _NSUB + k])

  def drain_scatter(j, b):
    for k in range(_NSUB):
      pltpu.make_async_copy(table.at[idx_v.at[j, 0, pl.ds(k * sub, sub)]],
                            rows_v.at[b, pl.ds(k * sub, sub)],
                            sems[b * _NSUB + k]).wait()
    pltpu.sync_copy(rows_v.at[b], agg_sh.at[idx_v.at[j, 1]], add=True)

  fire(0, 0)

  def _pair(p, carry):
    fire(2 * p + 1, 1)
    drain_scatter(2 * p, 0)

    @pl.when(2 * p + 2 < nch)
    def _():
      fire(2 * p + 2, 0)

    drain_scatter(2 * p + 1, 1)
    return carry

  lax.fori_loop(0, nch // 2, _pair, 0)
  plsc.subcore_barrier()

  pltpu.sync_copy(agg_sh.at[pl.ds(base, _RPT)],
                  agg_out.at[pl.ds(c * _N_PAD + base, _RPT)])


def _deg_body(dsti, zb128, ones128, deg_out, deg_sh, dst_v, ones_v, zb_v):
  c = lax.axis_index("c")
  s = lax.axis_index("s")
  wid = c * _NS + s
  pltpu.sync_copy(dsti.at[wid], dst_v)
  pltpu.sync_copy(zb128, zb_v)
  pltpu.sync_copy(ones128, ones_v)
  base = s * _RPT

  def _zero(t, carry):
    pltpu.sync_copy(zb_v, deg_sh.at[pl.ds(base + t * 8, 8)])
    return carry

  lax.fori_loop(0, _RPT // 8, _zero, 0)
  plsc.subcore_barrier()

  def _edge(j, carry):
    pltpu.sync_copy(ones_v, deg_sh.at[dst_v.at[j]], add=True)
    return carry

  lax.fori_loop(0, _NCHUNK, _edge, 0)
  plsc.subcore_barrier()
  pltpu.sync_copy(deg_sh.at[pl.ds(base, _RPT)],
                  deg_out.at[pl.ds(c * _N_PAD + base, _RPT)])


def _sc_mesh():
  return plsc.VectorSubcoreMesh(core_axis_name="c", subcore_axis_name="s",
                                num_cores=_NC, num_subcores=_NS)


_sc_cache = {}


def _get_agg():
  if "agg" not in _sc_cache:
    _sc_cache["agg"] = pl.kernel(
        _agg_body,
        out_type=[jax.ShapeDtypeStruct((_NC * _N_PAD, _D), jnp.float32)],
        mesh=_sc_mesh(),
        scratch_types=[
            pltpu.VMEM_SHARED((_N_PAD, _D), jnp.float32),
            pltpu.VMEM((_CA, 2, _GC), jnp.int32),
            pltpu.VMEM((2, _GC, _D), jnp.float32),
            pltpu.VMEM((8, _D), jnp.float32),
        ] + [pltpu.SemaphoreType.DMA] * (2 * _NSUB),
    )
  return _sc_cache["agg"]


def _get_deg():
  if "deg" not in _sc_cache:
    _sc_cache["deg"] = pl.kernel(
        _deg_body,
        out_type=[jax.ShapeDtypeStruct((_NC * _N_PAD, _D), jnp.float32)],
        mesh=_sc_mesh(),
        scratch_types=[
            pltpu.VMEM_SHARED((_N_PAD, _D), jnp.float32),
            pltpu.VMEM((_NCHUNK, _CHUNK), jnp.int32),
            pltpu.VMEM((_CHUNK, _D), jnp.float32),
            pltpu.VMEM((8, _D), jnp.float32),
        ],
    )
  return _sc_cache["deg"]


_BN = 1264


def _mm_body(x_ref, w_ref, o_ref):
  o_ref[...] = jnp.dot(x_ref[...], w_ref[...],
                       preferred_element_type=jnp.float32)


def _tc_table(h_pad, wcat):
  return pl.pallas_call(
      _mm_body,
      grid=(_N_PAD // _BN,),
      in_specs=[
          pl.BlockSpec((_BN, _D), lambda i: (i, 0)),
          pl.BlockSpec((_D, _R * _D), lambda i: (0, 0)),
      ],
      out_specs=pl.BlockSpec((_BN, _R * _D), lambda i: (i, 0)),
      out_shape=jax.ShapeDtypeStruct((_N_PAD, _R * _D), jnp.float32),
  )(h_pad, wcat)


def _layer_body(h_ref, a0_ref, a1_ref, d0_ref, d1_ref, lw_ref, b_ref,
                wc_ref, h1_ref, ht_ref):
  agg = a0_ref[...] + a1_ref[...]
  deg = d0_ref[:, 0:1] + d1_ref[:, 0:1]
  norm = jnp.where(deg > 0.0, 1.0 / jnp.maximum(deg, 1.0), 0.0)
  z = agg * norm + jnp.dot(h_ref[...], lw_ref[...],
                           preferred_element_type=jnp.float32) + b_ref[...]
  h1 = jnp.maximum(z, 0.0)
  h1_ref[...] = h1
  ht_ref[...] = jnp.dot(h1, wc_ref[...], preferred_element_type=jnp.float32)


def _tc_layer_mid(h_pad, a0, a1, d0, d1, loop_w, b, wcat_next):
  return pl.pallas_call(
      _layer_body,
      grid=(_N_PAD // _BN,),
      in_specs=[
          pl.BlockSpec((_BN, _D), lambda i: (i, 0)),
          pl.BlockSpec((_BN, _D), lambda i: (i, 0)),
          pl.BlockSpec((_BN, _D), lambda i: (i, 0)),
          pl.BlockSpec((_BN, _D), lambda i: (i, 0)),
          pl.BlockSpec((_BN, _D), lambda i: (i, 0)),
          pl.BlockSpec((_D, _D), lambda i: (0, 0)),
          pl.BlockSpec((1, _D), lambda i: (0, 0)),
          pl.BlockSpec((_D, _R * _D), lambda i: (0, 0)),
      ],
      out_specs=[
          pl.BlockSpec((_BN, _D), lambda i: (i, 0)),
          pl.BlockSpec((_BN, _R * _D), lambda i: (i, 0)),
      ],
      out_shape=[
          jax.ShapeDtypeStruct((_N_PAD, _D), jnp.float32),
          jax.ShapeDtypeStruct((_N_PAD, _R * _D), jnp.float32),
      ],
  )(h_pad, a0, a1, d0, d1, loop_w, b, wcat_next)


def _final_body(h_ref, a0_ref, a1_ref, d0_ref, d1_ref, lw_ref, b_ref, o_ref):
  agg = a0_ref[...] + a1_ref[...]
  deg = d0_ref[:, 0:1] + d1_ref[:, 0:1]
  norm = jnp.where(deg > 0.0, 1.0 / jnp.maximum(deg, 1.0), 0.0)
  o_ref[...] = agg * norm + jnp.dot(h_ref[...], lw_ref[...],
                                    preferred_element_type=jnp.float32) + b_ref[...]


def _tc_layer_final(h_pad, a0, a1, d0, d1, loop_w, b):
  return pl.pallas_call(
      _final_body,
      grid=(_N_PAD // _BN,),
      in_specs=[
          pl.BlockSpec((_BN, _D), lambda i: (i, 0)),
          pl.BlockSpec((_BN, _D), lambda i: (i, 0)),
          pl.BlockSpec((_BN, _D), lambda i: (i, 0)),
          pl.BlockSpec((_BN, _D), lambda i: (i, 0)),
          pl.BlockSpec((_BN, _D), lambda i: (i, 0)),
          pl.BlockSpec((_D, _D), lambda i: (0, 0)),
          pl.BlockSpec((1, _D), lambda i: (0, 0)),
      ],
      out_specs=pl.BlockSpec((_BN, _D), lambda i: (i, 0)),
      out_shape=jax.ShapeDtypeStruct((_N_PAD, _D), jnp.float32),
  )(h_pad, a0, a1, d0, d1, loop_w, b)


def _blockdiag_cat(W):
  Wd = jnp.zeros((_R, _D, _D), W.dtype)
  for b in range(_NB):
    Wd = Wd.at[:, b * _SUB:(b + 1) * _SUB, b * _SUB:(b + 1) * _SUB].set(W[:, b])
  return Wd.transpose(1, 0, 2).reshape(_D, _R * _D)


@jax.jit
def kernel(h, edge_index, e_feat, W0, loop_w0, b0, W1, loop_w1, b1):
  src = edge_index[0].astype(jnp.int32)
  dst = edge_index[1].astype(jnp.int32)
  ef = e_feat.astype(jnp.int32)

  pad = _E_PAD - _E
  gidx = jnp.pad(src * _R + ef, (0, pad)).reshape(_NCH_TOT, 1, _GC)
  dstp = jnp.pad(dst, (0, pad), constant_values=_N)
  dsti = dstp.reshape(_NCH_TOT, 1, _GC)
  gd = jnp.concatenate([gidx, dsti], axis=1)
  gd = jnp.pad(gd, ((0, _CA - _CB), (0, 0), (0, 0)))
  dsti3 = dstp.reshape(_NW, _NCHUNK, _CHUNK)

  zb128 = jnp.zeros((8, _D), jnp.float32)
  ones128 = jnp.ones((_CHUNK, _D), jnp.float32)

  h_pad = jnp.pad(h, ((0, _N_PAD - _N), (0, 0)))
  wcat0 = _blockdiag_cat(W0)
  wcat1 = _blockdiag_cat(W1)

  (degp,) = _get_deg()(dsti3, zb128, ones128)
  d0 = degp[:_N_PAD]
  d1 = degp[_N_PAD:]

  ht0 = _tc_table(h_pad, wcat0).reshape(_N_PAD * _R, _D)
  (aggp0,) = _get_agg()(ht0, gd, zb128)
  h1_pad, ht1 = _tc_layer_mid(h_pad, aggp0[:_N_PAD], aggp0[_N_PAD:], d0, d1,
                              loop_w0, b0.reshape(1, _D), wcat1)

  (aggp1,) = _get_agg()(ht1.reshape(_N_PAD * _R, _D), gd, zb128)
  out = _tc_layer_final(h1_pad, aggp1[:_N_PAD], aggp1[_N_PAD:], d0, d1,
                        loop_w1, b1.reshape(1, _D))
  return out[:_N]
